# baseline (speedup 1.0000x reference)
import functools

import jax
import jax.numpy as jnp
from jax import lax
from jax.experimental import pallas as pl
from jax.experimental.pallas import tpu as pltpu
from jax.experimental.pallas import tpu_sc as plsc

L = 16
NW = 32
ROW = 128
CHUNK_ROWS = 8
NPHASE = 64


def _make_kernel(n_obs, n_points, n_cams):
    assert n_obs % ROW == 0
    n_rows = n_obs // ROW
    rows_base = n_rows // NW
    rows_extra = n_rows % NW
    chunk_obs = CHUNK_ROWS * ROW
    groups_per_chunk = chunk_obs // L
    assert (rows_base + 1 + CHUNK_ROWS - 1) // CHUNK_ROWS <= NPHASE

    mesh = plsc.VectorSubcoreMesh(core_axis_name="c", subcore_axis_name="s")

    buf_t = [
        pltpu.VMEM((chunk_obs,), jnp.int32),
        pltpu.VMEM((chunk_obs,), jnp.int32),
        pltpu.VMEM((chunk_obs,), jnp.float32),
        pltpu.VMEM((chunk_obs,), jnp.float32),
        pltpu.VMEM((chunk_obs,), jnp.float32),
        pltpu.VMEM((chunk_obs,), jnp.float32),
        pltpu.VMEM((chunk_obs,), jnp.float32),
        pltpu.SemaphoreType.DMA,
        pltpu.SemaphoreType.DMA,
        pltpu.SemaphoreType.DMA,
    ]

    @functools.partial(
        pl.kernel,
        mesh=mesh,
        compiler_params=pltpu.CompilerParams(needs_layout_passes=False),
        out_type=(jax.ShapeDtypeStruct((n_obs,), jnp.float32),
                  jax.ShapeDtypeStruct((n_obs,), jnp.float32)),
        scratch_types=[pltpu.VMEM((n_cams * 10,), jnp.float32)] + buf_t * 2,
    )
    def residual_kernel(cidx_hbm, pidx_hbm, ptx_hbm, pty_hbm, ptz_hbm,
                        cam_hbm, outx_hbm, outy_hbm, cam_v, *bufs):
        A, B = bufs[:10], bufs[10:]
        w = lax.axis_index("s") * 2 + lax.axis_index("c")
        my_rows = rows_base + jnp.where(w < rows_extra, 1, 0)
        row_base = rows_base * w + jnp.minimum(w, rows_extra)

        pltpu.sync_copy(cam_hbm, cam_v)

        two = jnp.float32(2.0)

        def base_ob(c):
            rb = row_base + jnp.minimum(c * CHUNK_ROWS, my_rows - CHUNK_ROWS)
            return rb * ROW

        def lin_issue(c, b):
            ob = base_ob(c)
            pltpu.async_copy(pidx_hbm.at[pl.ds(ob, chunk_obs)], b[0], b[7])
            pltpu.async_copy(cidx_hbm.at[pl.ds(ob, chunk_obs)], b[1], b[7])

        def lin_wait(b):
            pltpu.make_async_copy(
                pidx_hbm.at[pl.ds(0, chunk_obs)], b[0], b[7]).wait()
            pltpu.make_async_copy(
                cidx_hbm.at[pl.ds(0, chunk_obs)], b[1], b[7]).wait()

        def gather_fire(b):
            handles = []
            for j in range(CHUNK_ROWS):
                sl = pl.ds(j * ROW, ROW)
                idx = b[0].at[sl]
                handles.append(
                    pltpu.async_copy(ptx_hbm.at[idx], b[2].at[sl], b[8]))
                handles.append(
                    pltpu.async_copy(pty_hbm.at[idx], b[3].at[sl], b[8]))
                handles.append(
                    pltpu.async_copy(ptz_hbm.at[idx], b[4].at[sl], b[8]))
            return handles

        def out_issue(c, b):
            ob = base_ob(c)
            pltpu.async_copy(b[5], outx_hbm.at[pl.ds(ob, chunk_obs)], b[9])
            pltpu.async_copy(b[6], outy_hbm.at[pl.ds(ob, chunk_obs)], b[9])

        def out_wait(b):
            pltpu.make_async_copy(
                b[5], outx_hbm.at[pl.ds(0, chunk_obs)], b[9]).wait()
            pltpu.make_async_copy(
                b[6], outy_hbm.at[pl.ds(0, chunk_obs)], b[9]).wait()

        def compute_chunk(b):
            def do_group(g, carry):
                sl = pl.ds(g * L, L)
                ci10 = b[1][sl] * 10

                px = b[2][sl]
                py = b[3][sl]
                pz = b[4][sl]

                t0 = plsc.load_gather(cam_v, [ci10])
                t1 = plsc.load_gather(cam_v, [ci10 + 1])
                t2 = plsc.load_gather(cam_v, [ci10 + 2])
                qx = plsc.load_gather(cam_v, [ci10 + 3])
                qy = plsc.load_gather(cam_v, [ci10 + 4])
                qz = plsc.load_gather(cam_v, [ci10 + 5])
                qw = plsc.load_gather(cam_v, [ci10 + 6])
                fo = plsc.load_gather(cam_v, [ci10 + 7])
                k1 = plsc.load_gather(cam_v, [ci10 + 8])
                k2 = plsc.load_gather(cam_v, [ci10 + 9])

                uvx = qy * pz - qz * py
                uvy = qz * px - qx * pz
                uvz = qx * py - qy * px
                uuvx = qy * uvz - qz * uvy
                uuvy = qz * uvx - qx * uvz
                uuvz = qx * uvy - qy * uvx
                cpx = px + two * (qw * uvx + uuvx) + t0
                cpy = py + two * (qw * uvy + uuvy) + t1
                cpz = pz + two * (qw * uvz + uuvz) + t2

                inv = jnp.float32(-1.0) / cpz
                nx = cpx * inv
                ny = cpy * inv
                r2 = nx * nx + ny * ny
                dist = jnp.float32(1.0) + r2 * (k1 + r2 * k2)
                fd = fo * dist

                b[5][sl] = fd * nx
                b[6][sl] = fd * ny
                return carry

            lax.fori_loop(0, groups_per_chunk, do_group, 0)

        def phase(c, cur, nxt, wait_out, comp):
            lin_wait(cur)
            handles = gather_fire(cur)
            lin_issue(c + 1, nxt)
            if wait_out:
                out_wait(nxt)
            if comp:
                compute_chunk(nxt)
                out_issue(c - 1, nxt)
            for h in handles:
                h.wait()

        lin_issue(0, A)
        phase(jnp.int32(0), A, B, False, False)
        phase(jnp.int32(1), B, A, False, True)
        phase(jnp.int32(2), A, B, False, True)
        phase(jnp.int32(3), B, A, True, True)

        def pair(i, carry):
            c = 2 * i
            phase(c, A, B, True, True)
            phase(c + 1, B, A, True, True)
            return carry

        lax.fori_loop(2, NPHASE // 2, pair, 0)

        lin_wait(A)
        out_wait(B)
        compute_chunk(B)
        out_issue(jnp.int32(NPHASE - 1), B)
        out_wait(A)
        out_wait(B)

    return residual_kernel


def kernel(observes, cidx, pidx, points, camera_params):
    n_obs = observes.shape[0]
    n_points, _ = points.shape
    n_cams, _ = camera_params.shape
    fn = _make_kernel(n_obs, n_points, n_cams)
    proj_x, proj_y = fn(cidx.astype(jnp.int32), pidx.astype(jnp.int32),
                        points[:, 0], points[:, 1], points[:, 2],
                        camera_params.reshape(-1))
    return jnp.stack([proj_x, proj_y], axis=-1) - observes


# speedup vs baseline: 1.0442x; 1.0442x over previous
import functools

import jax
import jax.numpy as jnp
from jax import lax
from jax.experimental import pallas as pl
from jax.experimental.pallas import tpu as pltpu
from jax.experimental.pallas import tpu_sc as plsc

L = 16
NW = 32
ROW = 128
CHUNK_ROWS = 16
NPHASE = 32


def _make_kernel(n_obs, n_points, n_cams):
    assert n_obs % ROW == 0
    n_rows = n_obs // ROW
    rows_base = n_rows // NW
    rows_extra = n_rows % NW
    chunk_obs = CHUNK_ROWS * ROW
    groups_per_chunk = chunk_obs // L
    assert (rows_base + 1 + CHUNK_ROWS - 1) // CHUNK_ROWS <= NPHASE

    mesh = plsc.VectorSubcoreMesh(core_axis_name="c", subcore_axis_name="s")

    buf_t = [
        pltpu.VMEM((chunk_obs,), jnp.int32),
        pltpu.VMEM((chunk_obs,), jnp.int32),
        pltpu.VMEM((chunk_obs,), jnp.float32),
        pltpu.VMEM((chunk_obs,), jnp.float32),
        pltpu.VMEM((chunk_obs,), jnp.float32),
        pltpu.VMEM((chunk_obs,), jnp.float32),
        pltpu.VMEM((chunk_obs,), jnp.float32),
        pltpu.SemaphoreType.DMA,
        pltpu.SemaphoreType.DMA,
        pltpu.SemaphoreType.DMA,
    ]

    @functools.partial(
        pl.kernel,
        mesh=mesh,
        compiler_params=pltpu.CompilerParams(needs_layout_passes=False),
        out_type=(jax.ShapeDtypeStruct((n_obs,), jnp.float32),
                  jax.ShapeDtypeStruct((n_obs,), jnp.float32)),
        scratch_types=[pltpu.VMEM((n_cams * 10,), jnp.float32)] + buf_t * 2,
    )
    def residual_kernel(cidx_hbm, pidx_hbm, ptx_hbm, pty_hbm, ptz_hbm,
                        cam_hbm, outx_hbm, outy_hbm, cam_v, *bufs):
        A, B = bufs[:10], bufs[10:]
        w = lax.axis_index("s") * 2 + lax.axis_index("c")
        my_rows = rows_base + jnp.where(w < rows_extra, 1, 0)
        row_base = rows_base * w + jnp.minimum(w, rows_extra)

        pltpu.sync_copy(cam_hbm, cam_v)

        two = jnp.float32(2.0)

        def base_ob(c):
            rb = row_base + jnp.minimum(c * CHUNK_ROWS, my_rows - CHUNK_ROWS)
            return rb * ROW

        def lin_issue(c, b):
            ob = base_ob(c)
            pltpu.async_copy(pidx_hbm.at[pl.ds(ob, chunk_obs)], b[0], b[7])
            pltpu.async_copy(cidx_hbm.at[pl.ds(ob, chunk_obs)], b[1], b[7])

        def lin_wait(b):
            pltpu.make_async_copy(
                pidx_hbm.at[pl.ds(0, chunk_obs)], b[0], b[7]).wait()
            pltpu.make_async_copy(
                cidx_hbm.at[pl.ds(0, chunk_obs)], b[1], b[7]).wait()

        def gather_fire(b):
            handles = []
            for j in range(CHUNK_ROWS):
                sl = pl.ds(j * ROW, ROW)
                idx = b[0].at[sl]
                handles.append(
                    pltpu.async_copy(ptx_hbm.at[idx], b[2].at[sl], b[8]))
                handles.append(
                    pltpu.async_copy(pty_hbm.at[idx], b[3].at[sl], b[8]))
                handles.append(
                    pltpu.async_copy(ptz_hbm.at[idx], b[4].at[sl], b[8]))
            return handles

        def out_issue(c, b):
            ob = base_ob(c)
            pltpu.async_copy(b[5], outx_hbm.at[pl.ds(ob, chunk_obs)], b[9])
            pltpu.async_copy(b[6], outy_hbm.at[pl.ds(ob, chunk_obs)], b[9])

        def out_wait(b):
            pltpu.make_async_copy(
                b[5], outx_hbm.at[pl.ds(0, chunk_obs)], b[9]).wait()
            pltpu.make_async_copy(
                b[6], outy_hbm.at[pl.ds(0, chunk_obs)], b[9]).wait()

        def compute_chunk(b):
            def do_group(g, carry):
                sl = pl.ds(g * L, L)
                ci10 = b[1][sl] * 10

                px = b[2][sl]
                py = b[3][sl]
                pz = b[4][sl]

                t0 = plsc.load_gather(cam_v, [ci10])
                t1 = plsc.load_gather(cam_v, [ci10 + 1])
                t2 = plsc.load_gather(cam_v, [ci10 + 2])
                qx = plsc.load_gather(cam_v, [ci10 + 3])
                qy = plsc.load_gather(cam_v, [ci10 + 4])
                qz = plsc.load_gather(cam_v, [ci10 + 5])
                qw = plsc.load_gather(cam_v, [ci10 + 6])
                fo = plsc.load_gather(cam_v, [ci10 + 7])
                k1 = plsc.load_gather(cam_v, [ci10 + 8])
                k2 = plsc.load_gather(cam_v, [ci10 + 9])

                uvx = qy * pz - qz * py
                uvy = qz * px - qx * pz
                uvz = qx * py - qy * px
                uuvx = qy * uvz - qz * uvy
                uuvy = qz * uvx - qx * uvz
                uuvz = qx * uvy - qy * uvx
                cpx = px + two * (qw * uvx + uuvx) + t0
                cpy = py + two * (qw * uvy + uuvy) + t1
                cpz = pz + two * (qw * uvz + uuvz) + t2

                inv = jnp.float32(-1.0) / cpz
                nx = cpx * inv
                ny = cpy * inv
                r2 = nx * nx + ny * ny
                dist = jnp.float32(1.0) + r2 * (k1 + r2 * k2)
                fd = fo * dist

                b[5][sl] = fd * nx
                b[6][sl] = fd * ny
                return carry

            lax.fori_loop(0, groups_per_chunk, do_group, 0)

        def phase(c, cur, nxt, wait_out, comp):
            lin_wait(cur)
            handles = gather_fire(cur)
            lin_issue(c + 1, nxt)
            if wait_out:
                out_wait(nxt)
            if comp:
                compute_chunk(nxt)
                out_issue(c - 1, nxt)
            for h in handles:
                h.wait()

        lin_issue(0, A)
        phase(jnp.int32(0), A, B, False, False)
        phase(jnp.int32(1), B, A, False, True)
        phase(jnp.int32(2), A, B, False, True)
        phase(jnp.int32(3), B, A, True, True)

        def pair(i, carry):
            c = 2 * i
            phase(c, A, B, True, True)
            phase(c + 1, B, A, True, True)
            return carry

        lax.fori_loop(2, NPHASE // 2, pair, 0)

        lin_wait(A)
        out_wait(B)
        compute_chunk(B)
        out_issue(jnp.int32(NPHASE - 1), B)
        out_wait(A)
        out_wait(B)

    return residual_kernel


def kernel(observes, cidx, pidx, points, camera_params):
    n_obs = observes.shape[0]
    n_points, _ = points.shape
    n_cams, _ = camera_params.shape
    fn = _make_kernel(n_obs, n_points, n_cams)
    proj_x, proj_y = fn(cidx.astype(jnp.int32), pidx.astype(jnp.int32),
                        points[:, 0], points[:, 1], points[:, 2],
                        camera_params.reshape(-1))
    return jnp.stack([proj_x, proj_y], axis=-1) - observes


# speedup vs baseline: 1.1627x; 1.1135x over previous
import functools

import jax
import jax.numpy as jnp
from jax import lax
from jax.experimental import pallas as pl
from jax.experimental.pallas import tpu as pltpu
from jax.experimental.pallas import tpu_sc as plsc

L = 16
NW = 32
ROW = 128
CHUNK_ROWS = 16
NPHASE = 32


def _make_kernel(n_obs, n_points, n_cams):
    assert n_obs % ROW == 0
    n_rows = n_obs // ROW
    rows_base = n_rows // NW
    rows_extra = n_rows % NW
    chunk_obs = CHUNK_ROWS * ROW
    groups_per_chunk = chunk_obs // L
    assert (rows_base + 1 + CHUNK_ROWS - 1) // CHUNK_ROWS <= NPHASE

    mesh = plsc.VectorSubcoreMesh(core_axis_name="c", subcore_axis_name="s")

    buf_t = [
        pltpu.VMEM((chunk_obs,), jnp.int32),
        pltpu.VMEM((chunk_obs,), jnp.int32),
        pltpu.VMEM((chunk_obs,), jnp.float32),
        pltpu.VMEM((chunk_obs,), jnp.float32),
        pltpu.VMEM((chunk_obs,), jnp.float32),
        pltpu.VMEM((chunk_obs * 2,), jnp.float32),
        pltpu.SemaphoreType.DMA,
        pltpu.SemaphoreType.DMA,
        pltpu.SemaphoreType.DMA,
    ]

    @functools.partial(
        pl.kernel,
        mesh=mesh,
        compiler_params=pltpu.CompilerParams(needs_layout_passes=False),
        out_type=jax.ShapeDtypeStruct((n_obs * 2,), jnp.float32),
        scratch_types=[pltpu.VMEM((n_cams * 10,), jnp.float32)] + buf_t * 2,
    )
    def residual_kernel(cidx_hbm, pidx_hbm, ptx_hbm, pty_hbm, ptz_hbm,
                        cam_hbm, out_hbm, cam_v, *bufs):
        A, B = bufs[:9], bufs[9:]
        w = lax.axis_index("s") * 2 + lax.axis_index("c")
        my_rows = rows_base + jnp.where(w < rows_extra, 1, 0)
        row_base = rows_base * w + jnp.minimum(w, rows_extra)

        pltpu.sync_copy(cam_hbm, cam_v)

        two = jnp.float32(2.0)

        def base_ob(c):
            rb = row_base + jnp.minimum(c * CHUNK_ROWS, my_rows - CHUNK_ROWS)
            return rb * ROW

        def lin_issue(c, b):
            ob = base_ob(c)
            pltpu.async_copy(pidx_hbm.at[pl.ds(ob, chunk_obs)], b[0], b[6])
            pltpu.async_copy(cidx_hbm.at[pl.ds(ob, chunk_obs)], b[1], b[6])

        def lin_wait(b):
            pltpu.make_async_copy(
                pidx_hbm.at[pl.ds(0, chunk_obs)], b[0], b[6]).wait()
            pltpu.make_async_copy(
                cidx_hbm.at[pl.ds(0, chunk_obs)], b[1], b[6]).wait()

        def gather_fire(b):
            handles = []
            for j in range(CHUNK_ROWS):
                sl = pl.ds(j * ROW, ROW)
                idx = b[0].at[sl]
                handles.append(
                    pltpu.async_copy(ptx_hbm.at[idx], b[2].at[sl], b[7]))
                handles.append(
                    pltpu.async_copy(pty_hbm.at[idx], b[3].at[sl], b[7]))
                handles.append(
                    pltpu.async_copy(ptz_hbm.at[idx], b[4].at[sl], b[7]))
            return handles

        def out_issue(c, b):
            ob = base_ob(c)
            pltpu.async_copy(
                b[5], out_hbm.at[pl.ds(ob * 2, chunk_obs * 2)], b[8])

        def out_wait(b):
            pltpu.make_async_copy(
                b[5], out_hbm.at[pl.ds(0, chunk_obs * 2)], b[8]).wait()

        def compute_chunk(b):
            def do_group(g, carry):
                sl = pl.ds(g * L, L)
                ci10 = b[1][sl] * 10

                px = b[2][sl]
                py = b[3][sl]
                pz = b[4][sl]

                t0 = plsc.load_gather(cam_v, [ci10])
                t1 = plsc.load_gather(cam_v, [ci10 + 1])
                t2 = plsc.load_gather(cam_v, [ci10 + 2])
                qx = plsc.load_gather(cam_v, [ci10 + 3])
                qy = plsc.load_gather(cam_v, [ci10 + 4])
                qz = plsc.load_gather(cam_v, [ci10 + 5])
                qw = plsc.load_gather(cam_v, [ci10 + 6])
                fo = plsc.load_gather(cam_v, [ci10 + 7])
                k1 = plsc.load_gather(cam_v, [ci10 + 8])
                k2 = plsc.load_gather(cam_v, [ci10 + 9])

                uvx = qy * pz - qz * py
                uvy = qz * px - qx * pz
                uvz = qx * py - qy * px
                uuvx = qy * uvz - qz * uvy
                uuvy = qz * uvx - qx * uvz
                uuvz = qx * uvy - qy * uvx
                cpx = px + two * (qw * uvx + uuvx) + t0
                cpy = py + two * (qw * uvy + uuvy) + t1
                cpz = pz + two * (qw * uvz + uuvz) + t2

                inv = jnp.float32(-1.0) / cpz
                nx = cpx * inv
                ny = cpy * inv
                r2 = nx * nx + ny * ny
                dist = jnp.float32(1.0) + r2 * (k1 + r2 * k2)
                fd = fo * dist

                off = (g >> 3) * (2 * ROW) + (g & 7) * L
                b[5][pl.ds(off, L)] = fd * nx
                b[5][pl.ds(off + ROW, L)] = fd * ny
                return carry

            lax.fori_loop(0, groups_per_chunk, do_group, 0)

        def phase(c, cur, nxt, wait_out, comp):
            lin_wait(cur)
            handles = gather_fire(cur)
            lin_issue(c + 1, nxt)
            if wait_out:
                out_wait(nxt)
            if comp:
                compute_chunk(nxt)
                out_issue(c - 1, nxt)
            for h in handles:
                h.wait()

        lin_issue(0, A)
        phase(jnp.int32(0), A, B, False, False)
        phase(jnp.int32(1), B, A, False, True)
        phase(jnp.int32(2), A, B, False, True)
        phase(jnp.int32(3), B, A, True, True)

        def pair(i, carry):
            c = 2 * i
            phase(c, A, B, True, True)
            phase(c + 1, B, A, True, True)
            return carry

        lax.fori_loop(2, NPHASE // 2, pair, 0)

        lin_wait(A)
        out_wait(B)
        compute_chunk(B)
        out_issue(jnp.int32(NPHASE - 1), B)
        out_wait(A)
        out_wait(B)

    return residual_kernel


def kernel(observes, cidx, pidx, points, camera_params):
    n_obs = observes.shape[0]
    n_points, _ = points.shape
    n_cams, _ = camera_params.shape
    fn = _make_kernel(n_obs, n_points, n_cams)
    proj = fn(cidx.astype(jnp.int32), pidx.astype(jnp.int32),
              points[:, 0], points[:, 1], points[:, 2],
              camera_params.reshape(-1))
    proj2 = proj.reshape(n_obs // ROW, 2, ROW).swapaxes(1, 2)
    return proj2.reshape(n_obs, 2) - observes


# speedup vs baseline: 1.1712x; 1.0073x over previous
import functools

import jax
import jax.numpy as jnp
from jax import lax
from jax.experimental import pallas as pl
from jax.experimental.pallas import tpu as pltpu
from jax.experimental.pallas import tpu_sc as plsc

L = 16
NW = 32
ROW = 128
CHUNK_ROWS = 16
NPHASE = 32


def _make_kernel(n_obs, n_points, n_cams):
    assert n_obs % ROW == 0
    n_rows = n_obs // ROW
    rows_base = n_rows // NW
    rows_extra = n_rows % NW
    chunk_obs = CHUNK_ROWS * ROW
    groups_per_chunk = chunk_obs // L
    assert (rows_base + 1 + CHUNK_ROWS - 1) // CHUNK_ROWS <= NPHASE

    mesh = plsc.VectorSubcoreMesh(core_axis_name="c", subcore_axis_name="s")

    buf_t = [
        pltpu.VMEM((chunk_obs,), jnp.int32),
        pltpu.VMEM((chunk_obs,), jnp.int32),
        pltpu.VMEM((chunk_obs,), jnp.float32),
        pltpu.VMEM((chunk_obs,), jnp.float32),
        pltpu.VMEM((chunk_obs,), jnp.float32),
        pltpu.VMEM((chunk_obs * 2,), jnp.float32),
        pltpu.SemaphoreType.DMA,
        pltpu.SemaphoreType.DMA,
        pltpu.SemaphoreType.DMA,
    ]

    @functools.partial(
        pl.kernel,
        mesh=mesh,
        compiler_params=pltpu.CompilerParams(needs_layout_passes=False),
        out_type=jax.ShapeDtypeStruct((n_obs * 2,), jnp.float32),
        scratch_types=[pltpu.VMEM((n_cams * 10,), jnp.float32)] + buf_t * 2,
    )
    def residual_kernel(cidx_hbm, pidx_hbm, ptx_hbm, pty_hbm, ptz_hbm,
                        cam_hbm, out_hbm, cam_v, *bufs):
        A, B = bufs[:9], bufs[9:]
        w = lax.axis_index("s") * 2 + lax.axis_index("c")
        my_rows = rows_base + jnp.where(w < rows_extra, 1, 0)
        row_base = rows_base * w + jnp.minimum(w, rows_extra)

        pltpu.sync_copy(cam_hbm, cam_v)

        two = jnp.float32(2.0)

        def base_ob(c):
            rb = row_base + jnp.minimum(c * CHUNK_ROWS, my_rows - CHUNK_ROWS)
            return rb * ROW

        def lin_issue(c, b):
            ob = base_ob(c)
            pltpu.async_copy(pidx_hbm.at[pl.ds(ob, chunk_obs)], b[0], b[6])
            pltpu.async_copy(cidx_hbm.at[pl.ds(ob, chunk_obs)], b[1], b[6])

        def lin_wait(b):
            pltpu.make_async_copy(
                pidx_hbm.at[pl.ds(0, chunk_obs)], b[0], b[6]).wait()
            pltpu.make_async_copy(
                cidx_hbm.at[pl.ds(0, chunk_obs)], b[1], b[6]).wait()

        def gather_fire(b):
            handles = []
            for j in range(CHUNK_ROWS):
                sl = pl.ds(j * ROW, ROW)
                idx = b[0].at[sl]
                handles.append(
                    pltpu.async_copy(ptx_hbm.at[idx], b[2].at[sl], b[7]))
                handles.append(
                    pltpu.async_copy(pty_hbm.at[idx], b[3].at[sl], b[7]))
                handles.append(
                    pltpu.async_copy(ptz_hbm.at[idx], b[4].at[sl], b[7]))
            return handles

        def out_issue(c, b):
            ob = base_ob(c)
            pltpu.async_copy(
                b[5], out_hbm.at[pl.ds(ob * 2, chunk_obs * 2)], b[8])

        def out_wait(b):
            pltpu.make_async_copy(
                b[5], out_hbm.at[pl.ds(0, chunk_obs * 2)], b[8]).wait()

        def compute_chunk(b):
            def do_group(g, carry):
                sl = pl.ds(g * L, L)
                ci10 = b[1][sl] * 10

                px = b[2][sl]
                py = b[3][sl]
                pz = b[4][sl]

                t0 = plsc.load_gather(cam_v, [ci10])
                t1 = plsc.load_gather(cam_v, [ci10 + 1])
                t2 = plsc.load_gather(cam_v, [ci10 + 2])
                qx = plsc.load_gather(cam_v, [ci10 + 3])
                qy = plsc.load_gather(cam_v, [ci10 + 4])
                qz = plsc.load_gather(cam_v, [ci10 + 5])
                qw = plsc.load_gather(cam_v, [ci10 + 6])
                fo = plsc.load_gather(cam_v, [ci10 + 7])
                k1 = plsc.load_gather(cam_v, [ci10 + 8])
                k2 = plsc.load_gather(cam_v, [ci10 + 9])

                uvx = qy * pz - qz * py
                uvy = qz * px - qx * pz
                uvz = qx * py - qy * px
                uuvx = qy * uvz - qz * uvy
                uuvy = qz * uvx - qx * uvz
                uuvz = qx * uvy - qy * uvx
                cpx = px + two * (qw * uvx + uuvx) + t0
                cpy = py + two * (qw * uvy + uuvy) + t1
                cpz = pz + two * (qw * uvz + uuvz) + t2

                nx = -(cpx / cpz)
                ny = -(cpy / cpz)
                r2 = nx * nx + ny * ny
                dist = jnp.float32(1.0) + k1 * r2 + k2 * (r2 * r2)
                fd = fo * dist

                off = (g >> 3) * (2 * ROW) + (g & 7) * L
                b[5][pl.ds(off, L)] = fd * nx
                b[5][pl.ds(off + ROW, L)] = fd * ny
                return carry

            lax.fori_loop(0, groups_per_chunk, do_group, 0)

        def phase(c, cur, nxt, wait_out, comp):
            lin_wait(cur)
            handles = gather_fire(cur)
            lin_issue(c + 1, nxt)
            if wait_out:
                out_wait(nxt)
            if comp:
                compute_chunk(nxt)
                out_issue(c - 1, nxt)
            for h in handles:
                h.wait()

        lin_issue(0, A)
        phase(jnp.int32(0), A, B, False, False)
        phase(jnp.int32(1), B, A, False, True)
        phase(jnp.int32(2), A, B, False, True)
        phase(jnp.int32(3), B, A, True, True)

        def pair(i, carry):
            c = 2 * i
            phase(c, A, B, True, True)
            phase(c + 1, B, A, True, True)
            return carry

        lax.fori_loop(2, NPHASE // 2, pair, 0)

        lin_wait(A)
        out_wait(B)
        compute_chunk(B)
        out_issue(jnp.int32(NPHASE - 1), B)
        out_wait(A)
        out_wait(B)

    return residual_kernel


def kernel(observes, cidx, pidx, points, camera_params):
    n_obs = observes.shape[0]
    n_points, _ = points.shape
    n_cams, _ = camera_params.shape
    fn = _make_kernel(n_obs, n_points, n_cams)
    proj = fn(cidx.astype(jnp.int32), pidx.astype(jnp.int32),
              points[:, 0], points[:, 1], points[:, 2],
              camera_params.reshape(-1))
    proj2 = proj.reshape(n_obs // ROW, 2, ROW).swapaxes(1, 2)
    return proj2.reshape(n_obs, 2) - observes
